# baseline (speedup 1.0000x reference)
import functools

import jax
import jax.numpy as jnp
from jax import lax
from jax.experimental import pallas as pl
from jax.experimental.pallas import tpu as pltpu
from jax.experimental.pallas import tpu_sc as plsc

T, B, D = 197, 32, 384
E = 8
K = 2
CAP = 49
CP = 56
H = D * 4
GH = D // 4
BA = 4
NBA = B // BA


def _route_one(xb, gw1, gb1, gw2, gb2, s_lt_t, blockones):
    g = jax.lax.dot_general(gw1, xb, (((1,), (1,)), ((), ())),
                            preferred_element_type=jnp.float32)
    g = jnp.maximum(g + gb1, 0.0)
    logits = jax.lax.dot_general(gw2, g, (((1,), (0,)), ((), ())),
                                 preferred_element_type=jnp.float32)
    logits = logits + gb2
    m = jnp.max(logits, axis=0, keepdims=True)
    p = jnp.exp(logits - m)
    gating = p / jnp.sum(p, axis=0, keepdims=True)

    ge = gating[:, None, :]
    gf = gating[None, :, :]
    f_lt_e = (jax.lax.broadcasted_iota(jnp.int32, (E, E, T), 1)
              < jax.lax.broadcasted_iota(jnp.int32, (E, E, T), 0))
    rank_e = (jnp.sum((gf > ge).astype(jnp.int32), axis=1)
              + jnp.sum(((gf == ge) & f_lt_e).astype(jnp.int32), axis=1))
    chosen = gating * (rank_e < K).astype(jnp.float32)

    ct = jnp.transpose(chosen)
    tiles = []
    for e in range(E):
        vs = jnp.broadcast_to(chosen[e:e + 1, :], (T, T))
        vt = jnp.broadcast_to(ct[:, e:e + 1], (T, T))
        tiles.append(((vs > vt) | ((vs == vt) & s_lt_t)).astype(jnp.float32))
    ahead_all = jnp.concatenate(tiles, axis=1)
    rank_t = jax.lax.dot_general(ahead_all, blockones,
                                 (((1,), (0,)), ((), ())),
                                 preferred_element_type=jnp.float32)
    rank_c = jnp.transpose(rank_t)
    sel = rank_c < float(CAP)
    slot = jnp.where(sel, rank_c, 1000.0).astype(jnp.int32)
    fw = chosen * sel.astype(jnp.float32)

    c_iota = jax.lax.broadcasted_iota(jnp.int32, (E, CP, T), 1)
    p8 = (slot[:, None, :] == c_iota).astype(jnp.float32)
    t_row = jax.lax.broadcasted_iota(jnp.int32, (1, T), 1).astype(jnp.float32)
    tok = jax.lax.dot_general(t_row, p8.reshape(E * CP, T),
                              (((1,), (1,)), ((), ())),
                              preferred_element_type=jnp.float32)
    return slot, fw, tok


def _routing_kernel(x_ref, gw1_ref, gb1_ref, gw2_ref, gb2_ref,
                    slot_ref, fw_ref, tok_ref):
    s_lt_t = (jax.lax.broadcasted_iota(jnp.int32, (T, T), 1)
              < jax.lax.broadcasted_iota(jnp.int32, (T, T), 0))
    blockones = (jax.lax.broadcasted_iota(jnp.int32, (E * T, E), 0) // T
                 == jax.lax.broadcasted_iota(jnp.int32, (E * T, E), 1)
                 ).astype(jnp.float32)
    gw1 = gw1_ref[...]
    gb1 = gb1_ref[...]
    gw2 = gw2_ref[...]
    gb2 = gb2_ref[...]
    for bl in range(BA):
        b_glob = pl.program_id(0) * BA + bl
        xb = x_ref[:, bl * D:(bl + 1) * D]
        slot, fw, tok = _route_one(xb, gw1, gb1, gw2, gb2, s_lt_t, blockones)
        slot_ref[bl * E:(bl + 1) * E] = slot.reshape(E, 1, T)
        fw_ref[bl * E:(bl + 1) * E] = fw.reshape(E, 1, T)
        tok_ref[bl] = tok.astype(jnp.int32) * B + b_glob


def _sc_gather_kernel(x_hbm, idx_hbm, xg_hbm, idx_v, rows0, rows1, gsem, wsem):
    info = plsc.get_sparse_core_info()
    wid = lax.axis_index("s") * info.num_cores + lax.axis_index("c")
    bufs = (rows0, rows1)
    pltpu.sync_copy(idx_hbm.at[wid, 0], idx_v)

    def gather(e, buf):
        return pltpu.async_copy(x_hbm.at[idx_v.at[pl.ds(e * CP, CP)]],
                                buf, gsem)

    g = gather(0, bufs[0])
    wprev = None
    for e in range(E):
        g.wait()
        w = pltpu.async_copy(bufs[e % 2], xg_hbm.at[wid, e], wsem)
        if wprev is not None:
            wprev.wait()
        if e + 1 < E:
            g = gather(e + 1, bufs[(e + 1) % 2])
        wprev = w
    wprev.wait()


def _sc_gather(xflat, tok):
    mesh = plsc.VectorSubcoreMesh(core_axis_name="c", subcore_axis_name="s")
    return pl.kernel(
        _sc_gather_kernel,
        mesh=mesh,
        out_type=jax.ShapeDtypeStruct((B, E, CP, D), jnp.float32),
        scratch_types=[
            pltpu.VMEM((E * CP,), jnp.int32),
            pltpu.VMEM((CP, D), jnp.float32),
            pltpu.VMEM((CP, D), jnp.float32),
            pltpu.SemaphoreType.DMA,
            pltpu.SemaphoreType.DMA,
        ],
    )(xflat, tok)


def _ffn_kernel(xg_ref, wfc_ref, bfc_ref, wpj_ref, bpj_ref, y_ref):
    xg = xg_ref[...].reshape(B * CP, D).astype(jnp.bfloat16)
    h = jax.lax.dot_general(xg, wfc_ref[0].astype(jnp.bfloat16),
                            (((1,), (1,)), ((), ())),
                            preferred_element_type=jnp.float32)
    h = jnp.maximum(h + bfc_ref[0], 0.0)
    y = jax.lax.dot_general(h.astype(jnp.bfloat16),
                            wpj_ref[0].astype(jnp.bfloat16),
                            (((1,), (1,)), ((), ())),
                            preferred_element_type=jnp.float32)
    y = y + bpj_ref[0]
    y_ref[...] = y.astype(jnp.bfloat16).reshape(B, 1, CP, D)


def _combine_kernel(y_ref, slot_ref, fw_ref, x_ref, out_ref):
    c_iota = jax.lax.broadcasted_iota(jnp.int32, (E, CP, T), 1)
    for bl in range(BA):
        slot = slot_ref[bl * E:(bl + 1) * E]
        fw = fw_ref[bl * E:(bl + 1) * E]
        w2t = jnp.where(slot == c_iota, fw, 0.0).astype(jnp.bfloat16)
        yb = y_ref[bl].reshape(E * CP, D)
        acc = jax.lax.dot_general(w2t.reshape(E * CP, T), yb,
                                  (((0,), (0,)), ((), ())),
                                  preferred_element_type=jnp.float32)
        out_ref[:, bl * D:(bl + 1) * D] = acc - x_ref[:, bl * D:(bl + 1) * D]


@jax.jit
def kernel(x, gW1, gb1, gW2, gb2, Wfc, bfc, Wproj, bproj):
    x2 = x.reshape(T, B * D)
    gb1c = gb1.reshape(GH, 1)
    gb2c = gb2.reshape(E, 1)
    bfc3 = bfc.reshape(E, 1, H)
    bpj3 = bproj.reshape(E, 1, D)

    slot, fw, tok = pl.pallas_call(
        _routing_kernel,
        grid=(NBA,),
        in_specs=[
            pl.BlockSpec((T, BA * D), lambda b: (0, b)),
            pl.BlockSpec((GH, D), lambda b: (0, 0)),
            pl.BlockSpec((GH, 1), lambda b: (0, 0)),
            pl.BlockSpec((E, GH), lambda b: (0, 0)),
            pl.BlockSpec((E, 1), lambda b: (0, 0)),
        ],
        out_specs=[
            pl.BlockSpec((BA * E, 1, T), lambda b: (b, 0, 0)),
            pl.BlockSpec((BA * E, 1, T), lambda b: (b, 0, 0)),
            pl.BlockSpec((BA, 1, E * CP), lambda b: (b, 0, 0)),
        ],
        out_shape=[
            jax.ShapeDtypeStruct((B * E, 1, T), jnp.int32),
            jax.ShapeDtypeStruct((B * E, 1, T), jnp.float32),
            jax.ShapeDtypeStruct((B, 1, E * CP), jnp.int32),
        ],
    )(x2, gW1, gb1c, gW2, gb2c)

    xg = _sc_gather(x.reshape(T * B, D), tok)

    y = pl.pallas_call(
        _ffn_kernel,
        grid=(E,),
        in_specs=[
            pl.BlockSpec((B, 1, CP, D), lambda e: (0, e, 0, 0)),
            pl.BlockSpec((1, H, D), lambda e: (e, 0, 0)),
            pl.BlockSpec((1, 1, H), lambda e: (e, 0, 0)),
            pl.BlockSpec((1, D, H), lambda e: (e, 0, 0)),
            pl.BlockSpec((1, 1, D), lambda e: (e, 0, 0)),
        ],
        out_specs=pl.BlockSpec((B, 1, CP, D), lambda e: (0, e, 0, 0)),
        out_shape=jax.ShapeDtypeStruct((B, E, CP, D), jnp.bfloat16),
    )(xg, Wfc, bfc3, Wproj, bpj3)

    out2 = pl.pallas_call(
        _combine_kernel,
        grid=(NBA,),
        in_specs=[
            pl.BlockSpec((BA, E, CP, D), lambda b: (b, 0, 0, 0)),
            pl.BlockSpec((BA * E, 1, T), lambda b: (b, 0, 0)),
            pl.BlockSpec((BA * E, 1, T), lambda b: (b, 0, 0)),
            pl.BlockSpec((T, BA * D), lambda b: (0, b)),
        ],
        out_specs=pl.BlockSpec((T, BA * D), lambda b: (0, b)),
        out_shape=jax.ShapeDtypeStruct((T, B * D), jnp.float32),
    )(y, slot, fw, x2)

    return out2.reshape(T, B, D)


# speedup vs baseline: 1.0267x; 1.0267x over previous
import functools

import jax
import jax.numpy as jnp
from jax import lax
from jax.experimental import pallas as pl
from jax.experimental.pallas import tpu as pltpu
from jax.experimental.pallas import tpu_sc as plsc

T, B, D = 197, 32, 384
E = 8
K = 2
CAP = 49
CP = 56
H = D * 4
GH = D // 4
BA = 4
NBA = B // BA


def _route_one(xb, gw1, gb1, gw2, gb2, s_lt_t):
    g = jax.lax.dot_general(gw1, xb, (((1,), (1,)), ((), ())),
                            preferred_element_type=jnp.float32)
    g = jnp.maximum(g + gb1, 0.0)
    logits = jax.lax.dot_general(gw2, g, (((1,), (0,)), ((), ())),
                                 preferred_element_type=jnp.float32)
    logits = logits + gb2
    m = jnp.max(logits, axis=0, keepdims=True)
    p = jnp.exp(logits - m)
    gating = p / jnp.sum(p, axis=0, keepdims=True)

    ge = gating[:, None, :]
    gf = gating[None, :, :]
    f_lt_e = (jax.lax.broadcasted_iota(jnp.int32, (E, E, T), 1)
              < jax.lax.broadcasted_iota(jnp.int32, (E, E, T), 0))
    rank_e = (jnp.sum((gf > ge).astype(jnp.int32), axis=1)
              + jnp.sum(((gf == ge) & f_lt_e).astype(jnp.int32), axis=1))
    chosen = gating * (rank_e < K).astype(jnp.float32)

    ct = jnp.transpose(chosen)
    ones_t = jnp.ones((T, 1), jnp.float32)
    cols = []
    for e in range(E):
        vs = jnp.broadcast_to(chosen[e:e + 1, :], (T, T))
        vt = jnp.broadcast_to(ct[:, e:e + 1], (T, T))
        ahead = ((vs > vt) | ((vs == vt) & s_lt_t)).astype(jnp.float32)
        cols.append(jax.lax.dot_general(ahead, ones_t, (((1,), (0,)), ((), ())),
                                        preferred_element_type=jnp.float32))
    rank_t = jnp.concatenate(cols, axis=1)
    rank_c = jnp.transpose(rank_t)
    sel = rank_c < float(CAP)
    slot = jnp.where(sel, rank_c, 1000.0).astype(jnp.int32)
    fw = chosen * sel.astype(jnp.float32)

    c_iota = jax.lax.broadcasted_iota(jnp.int32, (E, CP, T), 1)
    p8 = (slot[:, None, :] == c_iota).astype(jnp.float32)
    t_row = jax.lax.broadcasted_iota(jnp.int32, (1, T), 1).astype(jnp.float32)
    tok = jax.lax.dot_general(t_row, p8.reshape(E * CP, T),
                              (((1,), (1,)), ((), ())),
                              preferred_element_type=jnp.float32)
    return slot, fw, tok


def _routing_kernel(x_ref, gw1_ref, gb1_ref, gw2_ref, gb2_ref,
                    slot_ref, fw_ref, tok_ref):
    s_lt_t = (jax.lax.broadcasted_iota(jnp.int32, (T, T), 1)
              < jax.lax.broadcasted_iota(jnp.int32, (T, T), 0))
    gw1 = gw1_ref[...]
    gb1 = gb1_ref[...]
    gw2 = gw2_ref[...]
    gb2 = gb2_ref[...]
    for bl in range(BA):
        b_glob = pl.program_id(0) * BA + bl
        xb = x_ref[:, bl * D:(bl + 1) * D]
        slot, fw, tok = _route_one(xb, gw1, gb1, gw2, gb2, s_lt_t)
        slot_ref[bl * E:(bl + 1) * E] = slot.reshape(E, 1, T)
        fw_ref[bl * E:(bl + 1) * E] = fw.reshape(E, 1, T)
        tok_ref[bl] = tok.astype(jnp.int32) * B + b_glob


def _sc_gather_kernel(x_hbm, idx_hbm, xg_hbm, idx_v, rows0, rows1, gsem, wsem):
    info = plsc.get_sparse_core_info()
    wid = lax.axis_index("s") * info.num_cores + lax.axis_index("c")
    bufs = (rows0, rows1)
    pltpu.sync_copy(idx_hbm.at[wid, 0], idx_v)

    def gather(e, buf):
        return pltpu.async_copy(x_hbm.at[idx_v.at[pl.ds(e * CP, CP)]],
                                buf, gsem)

    g = gather(0, bufs[0])
    wprev = None
    for e in range(E):
        g.wait()
        w = pltpu.async_copy(bufs[e % 2], xg_hbm.at[wid, e], wsem)
        if wprev is not None:
            wprev.wait()
        if e + 1 < E:
            g = gather(e + 1, bufs[(e + 1) % 2])
        wprev = w
    wprev.wait()


def _sc_gather(xflat, tok):
    mesh = plsc.VectorSubcoreMesh(core_axis_name="c", subcore_axis_name="s")
    return pl.kernel(
        _sc_gather_kernel,
        mesh=mesh,
        out_type=jax.ShapeDtypeStruct((B, E, CP, D), jnp.float32),
        scratch_types=[
            pltpu.VMEM((E * CP,), jnp.int32),
            pltpu.VMEM((CP, D), jnp.float32),
            pltpu.VMEM((CP, D), jnp.float32),
            pltpu.SemaphoreType.DMA,
            pltpu.SemaphoreType.DMA,
        ],
    )(xflat, tok)


def _ffn_kernel(xg_ref, wfc_ref, bfc_ref, wpj_ref, bpj_ref, y_ref):
    xg = xg_ref[...].reshape(B * CP, D).astype(jnp.bfloat16)
    h = jax.lax.dot_general(xg, wfc_ref[0].astype(jnp.bfloat16),
                            (((1,), (1,)), ((), ())),
                            preferred_element_type=jnp.float32)
    h = jnp.maximum(h + bfc_ref[0], 0.0)
    y = jax.lax.dot_general(h.astype(jnp.bfloat16),
                            wpj_ref[0].astype(jnp.bfloat16),
                            (((1,), (1,)), ((), ())),
                            preferred_element_type=jnp.float32)
    y = y + bpj_ref[0]
    y_ref[...] = y.astype(jnp.bfloat16).reshape(B, 1, CP, D)


def _combine_kernel(y_ref, slot_ref, fw_ref, x_ref, out_ref):
    c_iota = jax.lax.broadcasted_iota(jnp.int32, (E, CP, T), 1)
    for bl in range(BA):
        slot = slot_ref[bl * E:(bl + 1) * E]
        fw = fw_ref[bl * E:(bl + 1) * E]
        w2t = jnp.where(slot == c_iota, fw, 0.0).astype(jnp.bfloat16)
        yb = y_ref[bl].reshape(E * CP, D)
        acc = jax.lax.dot_general(w2t.reshape(E * CP, T), yb,
                                  (((0,), (0,)), ((), ())),
                                  preferred_element_type=jnp.float32)
        out_ref[:, bl * D:(bl + 1) * D] = acc - x_ref[:, bl * D:(bl + 1) * D]


@jax.jit
def kernel(x, gW1, gb1, gW2, gb2, Wfc, bfc, Wproj, bproj):
    x2 = x.reshape(T, B * D)
    gb1c = gb1.reshape(GH, 1)
    gb2c = gb2.reshape(E, 1)
    bfc3 = bfc.reshape(E, 1, H)
    bpj3 = bproj.reshape(E, 1, D)

    slot, fw, tok = pl.pallas_call(
        _routing_kernel,
        grid=(NBA,),
        in_specs=[
            pl.BlockSpec((T, BA * D), lambda b: (0, b)),
            pl.BlockSpec((GH, D), lambda b: (0, 0)),
            pl.BlockSpec((GH, 1), lambda b: (0, 0)),
            pl.BlockSpec((E, GH), lambda b: (0, 0)),
            pl.BlockSpec((E, 1), lambda b: (0, 0)),
        ],
        out_specs=[
            pl.BlockSpec((BA * E, 1, T), lambda b: (b, 0, 0)),
            pl.BlockSpec((BA * E, 1, T), lambda b: (b, 0, 0)),
            pl.BlockSpec((BA, 1, E * CP), lambda b: (b, 0, 0)),
        ],
        out_shape=[
            jax.ShapeDtypeStruct((B * E, 1, T), jnp.int32),
            jax.ShapeDtypeStruct((B * E, 1, T), jnp.float32),
            jax.ShapeDtypeStruct((B, 1, E * CP), jnp.int32),
        ],
    )(x2, gW1, gb1c, gW2, gb2c)

    xg = _sc_gather(x.reshape(T * B, D), tok)

    y = pl.pallas_call(
        _ffn_kernel,
        grid=(E,),
        in_specs=[
            pl.BlockSpec((B, 1, CP, D), lambda e: (0, e, 0, 0)),
            pl.BlockSpec((1, H, D), lambda e: (e, 0, 0)),
            pl.BlockSpec((1, 1, H), lambda e: (e, 0, 0)),
            pl.BlockSpec((1, D, H), lambda e: (e, 0, 0)),
            pl.BlockSpec((1, 1, D), lambda e: (e, 0, 0)),
        ],
        out_specs=pl.BlockSpec((B, 1, CP, D), lambda e: (0, e, 0, 0)),
        out_shape=jax.ShapeDtypeStruct((B, E, CP, D), jnp.bfloat16),
    )(xg, Wfc, bfc3, Wproj, bpj3)

    out2 = pl.pallas_call(
        _combine_kernel,
        grid=(NBA,),
        in_specs=[
            pl.BlockSpec((BA, E, CP, D), lambda b: (b, 0, 0, 0)),
            pl.BlockSpec((BA * E, 1, T), lambda b: (b, 0, 0)),
            pl.BlockSpec((BA * E, 1, T), lambda b: (b, 0, 0)),
            pl.BlockSpec((T, BA * D), lambda b: (0, b)),
        ],
        out_specs=pl.BlockSpec((T, BA * D), lambda b: (0, b)),
        out_shape=jax.ShapeDtypeStruct((T, B * D), jnp.float32),
    )(y, slot, fw, x2)

    return out2.reshape(T, B, D)


# speedup vs baseline: 1.0283x; 1.0015x over previous
import jax
import jax.numpy as jnp
from jax import lax
from jax.experimental import pallas as pl
from jax.experimental.pallas import tpu as pltpu
from jax.experimental.pallas import tpu_sc as plsc

T, B, D = 197, 32, 384
E = 8
K = 2
CAP = 49
CP = 56
H = D * 4
GH = D // 4
BA = 4
NBA = B // BA


def _route_one(xb, gw1, gb1, gw2, gb2, s_lt_t):
    g = jax.lax.dot_general(gw1, xb, (((1,), (1,)), ((), ())),
                            preferred_element_type=jnp.float32)
    g = jnp.maximum(g + gb1, 0.0)
    logits = jax.lax.dot_general(gw2, g, (((1,), (0,)), ((), ())),
                                 preferred_element_type=jnp.float32)
    logits = logits + gb2
    m = jnp.max(logits, axis=0, keepdims=True)
    p = jnp.exp(logits - m)
    gating = p / jnp.sum(p, axis=0, keepdims=True)

    ge = gating[:, None, :]
    gf = gating[None, :, :]
    f_lt_e = (jax.lax.broadcasted_iota(jnp.int32, (E, E, T), 1)
              < jax.lax.broadcasted_iota(jnp.int32, (E, E, T), 0))
    rank_e = (jnp.sum((gf > ge).astype(jnp.int32), axis=1)
              + jnp.sum(((gf == ge) & f_lt_e).astype(jnp.int32), axis=1))
    chosen = gating * (rank_e < K).astype(jnp.float32)

    ct = jnp.transpose(chosen)
    ones_t = jnp.ones((T, 1), jnp.float32)
    cols = []
    for e in range(E):
        vs = jnp.broadcast_to(chosen[e:e + 1, :], (T, T))
        vt = jnp.broadcast_to(ct[:, e:e + 1], (T, T))
        ahead = ((vs > vt) | ((vs == vt) & s_lt_t)).astype(jnp.float32)
        cols.append(jax.lax.dot_general(ahead, ones_t, (((1,), (0,)), ((), ())),
                                        preferred_element_type=jnp.float32))
    rank_t = jnp.concatenate(cols, axis=1)
    rank_c = jnp.transpose(rank_t)
    sel = rank_c < float(CAP)
    slot = jnp.where(sel, rank_c, 1000.0).astype(jnp.int32)
    fw = chosen * sel.astype(jnp.float32)

    c_iota = jax.lax.broadcasted_iota(jnp.int32, (E, CP, T), 1)
    p8 = (slot[:, None, :] == c_iota).astype(jnp.float32)
    t_row = jax.lax.broadcasted_iota(jnp.int32, (1, T), 1).astype(jnp.float32)
    tok = jax.lax.dot_general(t_row, p8.reshape(E * CP, T),
                              (((1,), (1,)), ((), ())),
                              preferred_element_type=jnp.float32)
    return slot, fw, tok


def _routing_kernel(x_ref, gw1_ref, gb1_ref, gw2_ref, gb2_ref,
                    slot_ref, fw_ref, tok_ref):
    s_lt_t = (jax.lax.broadcasted_iota(jnp.int32, (T, T), 1)
              < jax.lax.broadcasted_iota(jnp.int32, (T, T), 0))
    gw1 = gw1_ref[...]
    gb1 = gb1_ref[...]
    gw2 = gw2_ref[...]
    gb2 = gb2_ref[...]
    for bl in range(BA):
        b_glob = pl.program_id(0) * BA + bl
        xb = x_ref[:, bl * D:(bl + 1) * D]
        slot, fw, tok = _route_one(xb, gw1, gb1, gw2, gb2, s_lt_t)
        slot_ref[bl * E:(bl + 1) * E] = slot.reshape(E, 1, T)
        fw_ref[bl * E:(bl + 1) * E] = fw.reshape(E, 1, T)
        tok_ref[bl] = tok.astype(jnp.int32) * B + b_glob


def _sc_gather_kernel(x_hbm, idx_hbm, xg_hbm, idx_v, rows0, rows1, gsem, wsem):
    info = plsc.get_sparse_core_info()
    wid = lax.axis_index("s") * info.num_cores + lax.axis_index("c")
    bufs = (rows0, rows1)
    pltpu.sync_copy(idx_hbm.at[wid, 0], idx_v)

    def gather(e, buf):
        return pltpu.async_copy(x_hbm.at[idx_v.at[pl.ds(e * CP, CP)]],
                                buf, gsem)

    g = gather(0, bufs[0])
    wprev = None
    for e in range(E):
        g.wait()
        w = pltpu.async_copy(bufs[e % 2], xg_hbm.at[wid, e], wsem)
        if wprev is not None:
            wprev.wait()
        if e + 1 < E:
            g = gather(e + 1, bufs[(e + 1) % 2])
        wprev = w
    wprev.wait()


def _sc_gather(xflat, tok):
    mesh = plsc.VectorSubcoreMesh(core_axis_name="c", subcore_axis_name="s")
    return pl.kernel(
        _sc_gather_kernel,
        mesh=mesh,
        out_type=jax.ShapeDtypeStruct((B, E, CP, D), jnp.float32),
        scratch_types=[
            pltpu.VMEM((E * CP,), jnp.int32),
            pltpu.VMEM((CP, D), jnp.float32),
            pltpu.VMEM((CP, D), jnp.float32),
            pltpu.SemaphoreType.DMA,
            pltpu.SemaphoreType.DMA,
        ],
    )(xflat, tok)


def _ffn_kernel(xg_ref, wfc_ref, bfc_ref, wpj_ref, bpj_ref, y_ref):
    xg = xg_ref[...].reshape(B * CP, D).astype(jnp.bfloat16)
    h = jax.lax.dot_general(xg, wfc_ref[0].astype(jnp.bfloat16),
                            (((1,), (1,)), ((), ())),
                            preferred_element_type=jnp.float32)
    h = jnp.maximum(h + bfc_ref[0], 0.0)
    y = jax.lax.dot_general(h.astype(jnp.bfloat16),
                            wpj_ref[0].astype(jnp.bfloat16),
                            (((1,), (1,)), ((), ())),
                            preferred_element_type=jnp.float32)
    y = y + bpj_ref[0]
    y_ref[...] = y.astype(jnp.bfloat16).reshape(B, 1, CP, D)


def _combine_kernel(y_ref, slot_ref, fw_ref, x_ref, out_ref):
    c_iota = jax.lax.broadcasted_iota(jnp.int32, (E, CP, T), 1)
    for bl in range(BA):
        slot = slot_ref[bl * E:(bl + 1) * E]
        fw = fw_ref[bl * E:(bl + 1) * E]
        w2t = jnp.where(slot == c_iota, fw, 0.0).astype(jnp.bfloat16)
        yb = y_ref[bl].reshape(E * CP, D)
        acc = jax.lax.dot_general(w2t.reshape(E * CP, T), yb,
                                  (((0,), (0,)), ((), ())),
                                  preferred_element_type=jnp.float32)
        out_ref[:, bl * D:(bl + 1) * D] = acc - x_ref[:, bl * D:(bl + 1) * D]


@jax.jit
def kernel(x, gW1, gb1, gW2, gb2, Wfc, bfc, Wproj, bproj):
    x2 = x.reshape(T, B * D)
    gb1c = gb1.reshape(GH, 1)
    gb2c = gb2.reshape(E, 1)
    bfc3 = bfc.reshape(E, 1, H)
    bpj3 = bproj.reshape(E, 1, D)

    slot, fw, tok = pl.pallas_call(
        _routing_kernel,
        grid=(NBA,),
        in_specs=[
            pl.BlockSpec((T, BA * D), lambda b: (0, b)),
            pl.BlockSpec((GH, D), lambda b: (0, 0)),
            pl.BlockSpec((GH, 1), lambda b: (0, 0)),
            pl.BlockSpec((E, GH), lambda b: (0, 0)),
            pl.BlockSpec((E, 1), lambda b: (0, 0)),
        ],
        out_specs=[
            pl.BlockSpec((BA * E, 1, T), lambda b: (b, 0, 0)),
            pl.BlockSpec((BA * E, 1, T), lambda b: (b, 0, 0)),
            pl.BlockSpec((BA, 1, E * CP), lambda b: (b, 0, 0)),
        ],
        out_shape=[
            jax.ShapeDtypeStruct((B * E, 1, T), jnp.int32),
            jax.ShapeDtypeStruct((B * E, 1, T), jnp.float32),
            jax.ShapeDtypeStruct((B, 1, E * CP), jnp.int32),
        ],
    )(x2, gW1, gb1c, gW2, gb2c)

    xg = _sc_gather(x.reshape(T * B, D), tok)

    y = pl.pallas_call(
        _ffn_kernel,
        grid=(E,),
        in_specs=[
            pl.BlockSpec((B, 1, CP, D), lambda e: (0, e, 0, 0)),
            pl.BlockSpec((1, H, D), lambda e: (e, 0, 0)),
            pl.BlockSpec((1, 1, H), lambda e: (e, 0, 0)),
            pl.BlockSpec((1, D, H), lambda e: (e, 0, 0)),
            pl.BlockSpec((1, 1, D), lambda e: (e, 0, 0)),
        ],
        out_specs=pl.BlockSpec((B, 1, CP, D), lambda e: (0, e, 0, 0)),
        out_shape=jax.ShapeDtypeStruct((B, E, CP, D), jnp.bfloat16),
    )(xg, Wfc, bfc3, Wproj, bpj3)

    out2 = pl.pallas_call(
        _combine_kernel,
        grid=(NBA,),
        in_specs=[
            pl.BlockSpec((BA, E, CP, D), lambda b: (b, 0, 0, 0)),
            pl.BlockSpec((BA * E, 1, T), lambda b: (b, 0, 0)),
            pl.BlockSpec((BA * E, 1, T), lambda b: (b, 0, 0)),
            pl.BlockSpec((T, BA * D), lambda b: (0, b)),
        ],
        out_specs=pl.BlockSpec((T, BA * D), lambda b: (0, b)),
        out_shape=jax.ShapeDtypeStruct((T, B * D), jnp.float32),
    )(y, slot, fw, x2)

    return out2.reshape(T, B, D)
